# baseline (speedup 1.0000x reference)
import functools

import jax
import jax.numpy as jnp
from jax import lax
from jax.experimental import pallas as pl
from jax.experimental.pallas import tpu as pltpu
from jax.experimental.pallas import tpu_sc as plsc

ROWS = 16384
FEATS = 26
EMB = 64
B = ROWS * FEATS
NC, NS = 2, 16
NW = NC * NS
B_PER_W = B // NW
CHUNK = 512
NCHUNK = B_PER_W // CHUNK
NPAIRS = NCHUNK // 2

_mesh = plsc.VectorSubcoreMesh(core_axis_name="c", subcore_axis_name="s")


@functools.partial(
    pl.kernel,
    mesh=_mesh,
    out_type=jax.ShapeDtypeStruct((B, EMB), jnp.float32),
    compiler_params=pltpu.CompilerParams(use_tc_tiling_on_sc=False),
    scratch_types=[
        pltpu.VMEM((B_PER_W,), jnp.int32),
        pltpu.VMEM((CHUNK, EMB), jnp.float32),
        pltpu.VMEM((CHUNK, EMB), jnp.float32),
        pltpu.SemaphoreType.DMA,
        pltpu.SemaphoreType.DMA,
    ],
)
def _gather_all(idx_hbm, table_hbm, out_hbm, idx_v, buf0, buf1, sem0, sem1):
    wid = lax.axis_index("s") * NC + lax.axis_index("c")
    base = wid * B_PER_W
    pltpu.sync_copy(idx_hbm.at[pl.ds(base, B_PER_W)], idx_v)

    pltpu.async_copy(table_hbm.at[idx_v.at[pl.ds(0, CHUNK)]], buf0, sem0)

    def pair(p, carry):
        off0 = 2 * p * CHUNK
        off1 = off0 + CHUNK
        pltpu.async_copy(table_hbm.at[idx_v.at[pl.ds(off1, CHUNK)]], buf1, sem1)
        pltpu.make_async_copy(
            table_hbm.at[idx_v.at[pl.ds(off0, CHUNK)]], buf0, sem0).wait()
        pltpu.sync_copy(buf0, out_hbm.at[pl.ds(base + off0, CHUNK)])

        @pl.when(p + 1 < NPAIRS)
        def _():
            pltpu.async_copy(
                table_hbm.at[idx_v.at[pl.ds(off1 + CHUNK, CHUNK)]], buf0, sem0)

        pltpu.make_async_copy(
            table_hbm.at[idx_v.at[pl.ds(off1, CHUNK)]], buf1, sem1).wait()
        pltpu.sync_copy(buf1, out_hbm.at[pl.ds(base + off1, CHUNK)])
        return carry

    lax.fori_loop(0, NPAIRS, pair, 0)


def kernel(node_label_index, W):
    idx = node_label_index.reshape(-1).astype(jnp.int32)
    out = _gather_all(idx, W)
    return out.reshape(ROWS, FEATS, EMB)
